# baseline (speedup 1.0000x reference)
import functools

import jax
import jax.numpy as jnp
from jax import lax
from jax.experimental import pallas as pl
from jax.experimental.pallas import tpu as pltpu
from jax.experimental.pallas import tpu_sc as plsc

N_NODES = 10000
D = 128
DH = D // 2
E = 320000
NC = 2
NS = 16
CHUNK = 128
NCH = 160
EPT = NCH * CHUNK
E_PAD = NS * EPT
NBUF = 4
PRO = 2
R = 10240
RPT = R // NS
ZCH = RPT // 128

_mesh = plsc.VectorSubcoreMesh(
    core_axis_name="c", subcore_axis_name="s", num_cores=NC, num_subcores=NS)
_sc_params = pltpu.CompilerParams(use_tc_tiling_on_sc=False)


def _seg_body(table, srcb, dstb, z64, sums_out, idx_s, idx_d, *rest):
    rows = rest[0:NBUF]
    zbuf, accum = rest[NBUF:NBUF + 2]
    sem_g = rest[NBUF + 2:NBUF + 2 + NBUF]
    sem_s = rest[NBUF + 2 + NBUF:NBUF + 2 + 2 * NBUF]

    c = lax.axis_index("c")
    s = lax.axis_index("s")
    wid = c * NS + s
    row0 = s * RPT

    pltpu.sync_copy(srcb.at[wid], idx_s)
    pltpu.sync_copy(dstb.at[wid], idx_d)

    pltpu.sync_copy(z64, zbuf)
    for z in range(ZCH):
        pltpu.sync_copy(zbuf, accum.at[pl.ds(row0 + z * 128, 128)])
    plsc.subcore_barrier()

    def g_issue(j, b):
        pltpu.async_copy(table.at[idx_s.at[j]], rows[b], sem_g[b])

    def g_wait(j, b):
        pltpu.make_async_copy(table.at[idx_s.at[j]], rows[b],
                              sem_g[b]).wait()

    def s_issue(j, b):
        pltpu.async_copy(rows[b], accum.at[idx_d.at[j]], sem_s[b],
                         add=True)

    def s_wait(j, b):
        pltpu.make_async_copy(rows[b], accum.at[idx_d.at[j]],
                              sem_s[b]).wait()

    for j in range(PRO):
        g_issue(j, j % NBUF)
    for j in range(PRO):
        g_wait(j, j % NBUF)
        s_issue(j, j % NBUF)
        g_issue(j + PRO, (j + PRO) % NBUF)

    n_outer = (NCH - 2 * PRO) // NBUF

    @pl.loop(0, n_outer)
    def _(jo):
        for b in range(NBUF):
            j = PRO + jo * NBUF + b
            bj = (PRO + b) % NBUF
            g_wait(j, bj)
            s_issue(j, bj)
            s_wait(j - PRO, b)
            g_issue(j + PRO, b)

    for t in range(PRO):
        j = NCH - PRO + t
        g_wait(j, j % NBUF)
        s_issue(j, j % NBUF)
        s_wait(j - PRO, (j - PRO) % NBUF)
    for t in range(PRO):
        j = NCH - PRO + t
        s_wait(j, j % NBUF)

    plsc.subcore_barrier()

    out0 = c * R + row0
    for z in range(ZCH):
        pltpu.sync_copy(accum.at[pl.ds(row0 + z * 128, 128)], zbuf)
        pltpu.sync_copy(zbuf, sums_out.at[pl.ds(out0 + z * 128, 128)])


_seg_sum = pl.kernel(
    _seg_body,
    out_type=jax.ShapeDtypeStruct((NC * R, DH), jnp.float32),
    mesh=_mesh,
    compiler_params=_sc_params,
    scratch_types=[
        pltpu.VMEM((NCH, CHUNK), jnp.int32),
        pltpu.VMEM((NCH, CHUNK), jnp.int32),
        *[pltpu.VMEM((CHUNK, DH), jnp.float32) for _ in range(NBUF)],
        pltpu.VMEM((128, DH), jnp.float32),
        pltpu.VMEM_SHARED((R, DH), jnp.float32),
        *[pltpu.SemaphoreType.DMA for _ in range(2 * NBUF)],
    ],
)

CSEM = 4


def _counts_body(dstb, z16, ones16, cnt_out, idx_d, cbuf, ones_v, accum_c,
                 sem):
    c = lax.axis_index("c")
    s = lax.axis_index("s")
    wid = c * NS + s
    row0 = s * RPT

    pltpu.sync_copy(dstb.at[wid], idx_d)
    pltpu.sync_copy(z16, cbuf)
    for z in range(ZCH):
        pltpu.sync_copy(cbuf, accum_c.at[pl.ds(row0 + z * 128, 128)])
    pltpu.sync_copy(ones16, ones_v)
    plsc.subcore_barrier()

    def s_issue(j):
        pltpu.async_copy(ones_v, accum_c.at[idx_d.at[j]], sem, add=True)

    def s_drain(j):
        pltpu.make_async_copy(ones_v, accum_c.at[idx_d.at[j]], sem).wait()

    for j in range(CSEM):
        s_issue(j)

    @pl.loop(CSEM, NCH)
    def _(j):
        s_drain(j - CSEM)
        s_issue(j)

    for t in range(CSEM):
        s_drain(NCH - CSEM + t)

    plsc.subcore_barrier()
    out0 = c * R + row0
    for z in range(ZCH):
        pltpu.sync_copy(accum_c.at[pl.ds(row0 + z * 128, 128)], cbuf)
        pltpu.sync_copy(cbuf, cnt_out.at[pl.ds(out0 + z * 128, 128)])


_counts = pl.kernel(
    _counts_body,
    out_type=jax.ShapeDtypeStruct((NC * R, 16), jnp.float32),
    mesh=_mesh,
    compiler_params=_sc_params,
    scratch_types=[
        pltpu.VMEM((NCH, CHUNK), jnp.int32),
        pltpu.VMEM((128, 16), jnp.float32),
        pltpu.VMEM((128, 16), jnp.float32),
        pltpu.VMEM_SHARED((R, 16), jnp.float32),
        pltpu.SemaphoreType.DMA,
    ],
)


def _dense_body(cnt_off, s_ref, c_ref, x_ref, wl_ref, wr_ref, b_ref, g_ref,
                bt_ref, o_ref):
    S = jnp.concatenate(
        [s_ref[0:N_NODES, :], s_ref[R:R + N_NODES, :]], axis=1)
    cnt = c_ref[cnt_off:cnt_off + N_NODES, 0:1]
    agg = S / jnp.maximum(cnt, 1.0)
    xu = (jnp.dot(agg, wl_ref[...], preferred_element_type=jnp.float32)
          + jnp.dot(x_ref[...], wr_ref[...], preferred_element_type=jnp.float32)
          + b_ref[...])
    m = jnp.mean(xu, axis=0, keepdims=True)
    d = xu - m
    v = jnp.mean(d * d, axis=0, keepdims=True)
    y = d * lax.rsqrt(v + 1e-5) * g_ref[...] + bt_ref[...]
    o_ref[...] = jnp.maximum(y, 0.0)


_dense_u = pl.pallas_call(
    functools.partial(_dense_body, 0),
    out_shape=jax.ShapeDtypeStruct((N_NODES, D), jnp.float32),
)
_dense_e = pl.pallas_call(
    functools.partial(_dense_body, R),
    out_shape=jax.ShapeDtypeStruct((N_NODES, D), jnp.float32),
)


def _as_blocks(idx, fill, bias):
    pad = jnp.full((E_PAD - E,), fill, jnp.int32)
    blk = jnp.concatenate([idx, pad]).reshape(1, NS, NCH, CHUNK)
    return jnp.concatenate([blk, blk + bias], axis=0).reshape(
        NC * NS, NCH, CHUNK)


def _split_cols(x):
    return jnp.concatenate([x[:, :DH], x[:, DH:]], axis=0)


def kernel(x_user, x_event, edge_index, params):
    u = edge_index[0].astype(jnp.int32)
    e = edge_index[1].astype(jnp.int32)
    src_u = _as_blocks(e, 0, N_NODES)
    dst_u = _as_blocks(u, R - 1, 0)
    src_e = _as_blocks(u, 0, N_NODES)
    dst_e = _as_blocks(e, R - 1, 0)
    dst_c = jnp.concatenate([dst_u[0:NS], dst_e[0:NS]], axis=0)

    z64 = jnp.zeros((128, DH), jnp.float32)
    z16 = jnp.zeros((128, 16), jnp.float32)
    ones16 = jnp.ones((128, 16), jnp.float32)

    def dense(fn, S, C, x, side, i):
        return fn(S, C, x,
                  params['Wl_%s%d' % (side, i)].T,
                  params['Wr_%s%d' % (side, i)].T,
                  params['bl_%s%d' % (side, i)].reshape(1, D),
                  params['gamma_%s%d' % (side, i)].reshape(1, D),
                  params['beta_%s%d' % (side, i)].reshape(1, D))

    cnt = _counts(dst_c, z16, ones16)

    Su = _seg_sum(_split_cols(x_event), src_u, dst_u, z64)
    x_user = dense(_dense_u, Su, cnt, x_user, 'u', 0)
    Se = _seg_sum(_split_cols(x_user), src_e, dst_e, z64)
    x_event = dense(_dense_e, Se, cnt, x_event, 'e', 0)

    Su2 = _seg_sum(_split_cols(x_event), src_u, dst_u, z64)
    x_user = dense(_dense_u, Su2, cnt, x_user, 'u', 1)
    Se2 = _seg_sum(_split_cols(x_user), src_e, dst_e, z64)
    x_event = dense(_dense_e, Se2, cnt, x_event, 'e', 1)
    return x_user, x_event


# speedup vs baseline: 1.0332x; 1.0332x over previous
import functools

import jax
import jax.numpy as jnp
from jax import lax
from jax.experimental import pallas as pl
from jax.experimental.pallas import tpu as pltpu
from jax.experimental.pallas import tpu_sc as plsc

N_NODES = 10000
D = 128
DH = D // 2
E = 320000
NC = 2
NS = 16
CHUNK = 128
NCH = 160
EPT = NCH * CHUNK
E_PAD = NS * EPT
NBUF = 4
PRO = 2
R = 10240
RPT = R // NS
ZCH = RPT // 128

_mesh = plsc.VectorSubcoreMesh(
    core_axis_name="c", subcore_axis_name="s", num_cores=NC, num_subcores=NS)
_sc_params = pltpu.CompilerParams(use_tc_tiling_on_sc=False)


def _seg_body(table, srcb, dstb, z64, sums_out, idx_s, idx_d, *rest):
    rows = rest[0:NBUF]
    zbuf, accum = rest[NBUF:NBUF + 2]
    sem_g = rest[NBUF + 2:NBUF + 2 + NBUF]
    sem_s = rest[NBUF + 2 + NBUF:NBUF + 2 + 2 * NBUF]

    c = lax.axis_index("c")
    s = lax.axis_index("s")
    wid = c * NS + s
    row0 = s * RPT

    pltpu.sync_copy(srcb.at[wid], idx_s)
    pltpu.sync_copy(dstb.at[wid], idx_d)

    pltpu.sync_copy(z64, zbuf)
    for z in range(ZCH):
        pltpu.sync_copy(zbuf, accum.at[pl.ds(row0 + z * 128, 128)])
    plsc.subcore_barrier()

    def g_issue(j, b):
        pltpu.async_copy(table.at[idx_s.at[j]], rows[b], sem_g[b])

    def g_wait(j, b):
        del j
        pltpu.make_async_copy(table.at[pl.ds(0, CHUNK)], rows[b],
                              sem_g[b]).wait()

    def s_issue(j, b):
        pltpu.async_copy(rows[b], accum.at[idx_d.at[j]], sem_s[b],
                         add=True)

    def s_wait(j, b):
        del j
        pltpu.make_async_copy(rows[b], accum.at[pl.ds(0, CHUNK)],
                              sem_s[b]).wait()

    for j in range(PRO):
        g_issue(j, j % NBUF)
    for j in range(PRO):
        g_wait(j, j % NBUF)
        s_issue(j, j % NBUF)
        g_issue(j + PRO, (j + PRO) % NBUF)

    n_outer = (NCH - 2 * PRO) // NBUF

    @pl.loop(0, n_outer)
    def _(jo):
        for b in range(NBUF):
            j = PRO + jo * NBUF + b
            bj = (PRO + b) % NBUF
            g_wait(j, bj)
            s_issue(j, bj)
            s_wait(j - PRO, b)
            g_issue(j + PRO, b)

    for t in range(PRO):
        j = NCH - PRO + t
        g_wait(j, j % NBUF)
        s_issue(j, j % NBUF)
        s_wait(j - PRO, (j - PRO) % NBUF)
    for t in range(PRO):
        j = NCH - PRO + t
        s_wait(j, j % NBUF)

    plsc.subcore_barrier()

    out0 = c * R + row0
    pltpu.sync_copy(accum.at[pl.ds(row0, RPT)],
                    sums_out.at[pl.ds(out0, RPT)])


_seg_sum = pl.kernel(
    _seg_body,
    out_type=jax.ShapeDtypeStruct((NC * R, DH), jnp.float32),
    mesh=_mesh,
    compiler_params=_sc_params,
    scratch_types=[
        pltpu.VMEM((NCH, CHUNK), jnp.int32),
        pltpu.VMEM((NCH, CHUNK), jnp.int32),
        *[pltpu.VMEM((CHUNK, DH), jnp.float32) for _ in range(NBUF)],
        pltpu.VMEM((128, DH), jnp.float32),
        pltpu.VMEM_SHARED((R, DH), jnp.float32),
        *[pltpu.SemaphoreType.DMA for _ in range(2 * NBUF)],
    ],
)

CSEM = 4


def _counts_body(dstb, z16, ones16, cnt_out, idx_d, cbuf, ones_v, accum_c,
                 sem):
    c = lax.axis_index("c")
    s = lax.axis_index("s")
    wid = c * NS + s
    row0 = s * RPT

    pltpu.sync_copy(dstb.at[wid], idx_d)
    pltpu.sync_copy(z16, cbuf)
    for z in range(ZCH):
        pltpu.sync_copy(cbuf, accum_c.at[pl.ds(row0 + z * 128, 128)])
    pltpu.sync_copy(ones16, ones_v)
    plsc.subcore_barrier()

    def s_issue(j):
        pltpu.async_copy(ones_v, accum_c.at[idx_d.at[j]], sem, add=True)

    def s_drain(j):
        pltpu.make_async_copy(ones_v, accum_c.at[idx_d.at[j]], sem).wait()

    for j in range(CSEM):
        s_issue(j)

    @pl.loop(CSEM, NCH)
    def _(j):
        s_drain(j - CSEM)
        s_issue(j)

    for t in range(CSEM):
        s_drain(NCH - CSEM + t)

    plsc.subcore_barrier()
    out0 = c * R + row0
    for z in range(ZCH):
        pltpu.sync_copy(accum_c.at[pl.ds(row0 + z * 128, 128)], cbuf)
        pltpu.sync_copy(cbuf, cnt_out.at[pl.ds(out0 + z * 128, 128)])


_counts = pl.kernel(
    _counts_body,
    out_type=jax.ShapeDtypeStruct((NC * R, 16), jnp.float32),
    mesh=_mesh,
    compiler_params=_sc_params,
    scratch_types=[
        pltpu.VMEM((NCH, CHUNK), jnp.int32),
        pltpu.VMEM((128, 16), jnp.float32),
        pltpu.VMEM((128, 16), jnp.float32),
        pltpu.VMEM_SHARED((R, 16), jnp.float32),
        pltpu.SemaphoreType.DMA,
    ],
)


def _dense_body(cnt_off, s_ref, c_ref, x_ref, wl_ref, wr_ref, b_ref, g_ref,
                bt_ref, o_ref):
    S = jnp.concatenate(
        [s_ref[0:N_NODES, :], s_ref[R:R + N_NODES, :]], axis=1)
    cnt = c_ref[cnt_off:cnt_off + N_NODES, 0:1]
    agg = S / jnp.maximum(cnt, 1.0)
    xu = (jnp.dot(agg, wl_ref[...], preferred_element_type=jnp.float32)
          + jnp.dot(x_ref[...], wr_ref[...], preferred_element_type=jnp.float32)
          + b_ref[...])
    m = jnp.mean(xu, axis=0, keepdims=True)
    d = xu - m
    v = jnp.mean(d * d, axis=0, keepdims=True)
    y = d * lax.rsqrt(v + 1e-5) * g_ref[...] + bt_ref[...]
    o_ref[...] = jnp.maximum(y, 0.0)


_dense_u = pl.pallas_call(
    functools.partial(_dense_body, 0),
    out_shape=jax.ShapeDtypeStruct((N_NODES, D), jnp.float32),
)
_dense_e = pl.pallas_call(
    functools.partial(_dense_body, R),
    out_shape=jax.ShapeDtypeStruct((N_NODES, D), jnp.float32),
)


def _as_blocks(idx, fill, bias):
    pad = jnp.full((E_PAD - E,), fill, jnp.int32)
    blk = jnp.concatenate([idx, pad]).reshape(1, NS, NCH, CHUNK)
    return jnp.concatenate([blk, blk + bias], axis=0).reshape(
        NC * NS, NCH, CHUNK)


def _split_cols(x):
    return jnp.concatenate([x[:, :DH], x[:, DH:]], axis=0)


def kernel(x_user, x_event, edge_index, params):
    u = edge_index[0].astype(jnp.int32)
    e = edge_index[1].astype(jnp.int32)
    src_u = _as_blocks(e, 0, N_NODES)
    dst_u = _as_blocks(u, R - 1, 0)
    src_e = _as_blocks(u, 0, N_NODES)
    dst_e = _as_blocks(e, R - 1, 0)
    dst_c = jnp.concatenate([dst_u[0:NS], dst_e[0:NS]], axis=0)

    z64 = jnp.zeros((128, DH), jnp.float32)
    z16 = jnp.zeros((128, 16), jnp.float32)
    ones16 = jnp.ones((128, 16), jnp.float32)

    def dense(fn, S, C, x, side, i):
        return fn(S, C, x,
                  params['Wl_%s%d' % (side, i)].T,
                  params['Wr_%s%d' % (side, i)].T,
                  params['bl_%s%d' % (side, i)].reshape(1, D),
                  params['gamma_%s%d' % (side, i)].reshape(1, D),
                  params['beta_%s%d' % (side, i)].reshape(1, D))

    cnt = _counts(dst_c, z16, ones16)

    Su = _seg_sum(_split_cols(x_event), src_u, dst_u, z64)
    x_user = dense(_dense_u, Su, cnt, x_user, 'u', 0)
    Se = _seg_sum(_split_cols(x_user), src_e, dst_e, z64)
    x_event = dense(_dense_e, Se, cnt, x_event, 'e', 0)

    Su2 = _seg_sum(_split_cols(x_event), src_u, dst_u, z64)
    x_user = dense(_dense_u, Su2, cnt, x_user, 'u', 1)
    Se2 = _seg_sum(_split_cols(x_user), src_e, dst_e, z64)
    x_event = dense(_dense_e, Se2, cnt, x_event, 'e', 1)
    return x_user, x_event
